# baseline (speedup 1.0000x reference)
import functools

import jax
import jax.numpy as jnp
from jax import lax
from jax.experimental import pallas as pl
from jax.experimental.pallas import tpu as pltpu
from jax.experimental.pallas import tpu_sc as plsc

B, L = 4096, 20
EMB = 64
PAIR = 2 * EMB
C = 235
N = B * L

NUM_CORES = 2
NUM_SUBCORES = 16
NW = NUM_CORES * NUM_SUBCORES
CHUNK = 128
PER_W = N // NW
NCH = PER_W // CHUNK

BBLK = 512
NB = B // BBLK


def _gather_body(table_hbm, idx_hbm, out_hbm, idx_v, rows, sems):
    wid = lax.axis_index("s") * NUM_CORES + lax.axis_index("c")
    out_row0 = wid * PER_W
    pltpu.sync_copy(idx_hbm.at[wid], idx_v)
    copies = [None, None]
    copies[0] = pltpu.async_copy(table_hbm.at[idx_v.at[0]], rows[0], sems[0])
    for j in range(NCH):
        cur = j % 2
        nxt = (j + 1) % 2
        if j + 1 < NCH:
            copies[nxt] = pltpu.async_copy(
                table_hbm.at[idx_v.at[j + 1]], rows[nxt], sems[nxt])
        copies[cur].wait()
        pltpu.sync_copy(rows[cur], out_hbm.at[pl.ds(out_row0 + j * CHUNK, CHUNK)])


@functools.cache
def _make_sc_gather():
    return pl.kernel(
        _gather_body,
        out_type=jax.ShapeDtypeStruct((N, PAIR), jnp.float32),
        mesh=plsc.VectorSubcoreMesh(core_axis_name="c", subcore_axis_name="s"),
        scratch_types=[
            pltpu.VMEM((NCH, CHUNK), jnp.int32),
            [pltpu.VMEM((CHUNK, PAIR), jnp.float32),
             pltpu.VMEM((CHUNK, PAIR), jnp.float32)],
            [pltpu.SemaphoreType.DMA, pltpu.SemaphoreType.DMA],
        ],
    )


def _half_select(pair_rows, par_col):
    return jnp.where(par_col == 1, pair_rows[:, EMB:], pair_rows[:, :EMB])


def _lse_kernel(emb_ref, par_ref, wt_ref, out_ref, m_ref, s_ref):
    i = pl.program_id(0)

    @pl.when(i == 0)
    def _init():
        m_ref[...] = jnp.full((L, C), -jnp.inf, dtype=jnp.float32)
        s_ref[...] = jnp.zeros((L, C), dtype=jnp.float32)

    e = emb_ref[...]
    par = par_ref[...]
    wt = wt_ref[...]
    for l in range(L):
        el = _half_select(e[:, l, :], par[:, l:l + 1])
        x = lax.dot_general(el, wt, (((1,), (0,)), ((), ())),
                            preferred_element_type=jnp.float32)
        bm = jnp.max(x, axis=0, keepdims=True)
        bs = jnp.sum(jnp.exp(x - bm), axis=0, keepdims=True)
        m_old = m_ref[pl.ds(l, 1), :]
        s_old = s_ref[pl.ds(l, 1), :]
        m_new = jnp.maximum(m_old, bm)
        s_ref[pl.ds(l, 1), :] = (s_old * jnp.exp(m_old - m_new)
                                 + bs * jnp.exp(bm - m_new))
        m_ref[pl.ds(l, 1), :] = m_new

    @pl.when(i == NB - 1)
    def _fin():
        out_ref[...] = m_ref[...] + jnp.log(s_ref[...])


def _out_kernel(emb_ref, par_ref, wt_ref, lse_ref, out_ref):
    e = emb_ref[...]
    par = par_ref[...]
    wt = wt_ref[...]
    lse = lse_ref[...]
    for l in range(L):
        el = _half_select(e[:, l, :], par[:, l:l + 1])
        x = lax.dot_general(el, wt, (((1,), (0,)), ((), ())),
                            preferred_element_type=jnp.float32)
        out_ref[:, pl.ds(l, 1), :] = (x - lse[l:l + 1, :])[:, None, :]


def kernel(input, emb_weight, fc_weight, fc_bias):
    idx = input.astype(jnp.int32)
    idx_pair = lax.shift_right_logical(idx, 1).reshape(NW, NCH, CHUNK)
    parity = lax.bitwise_and(idx, 1)
    table2 = emb_weight.reshape(-1, PAIR)

    pairs = _make_sc_gather()(table2, idx_pair)
    emb3 = pairs.reshape(B, L, PAIR)
    wt = fc_weight.T

    lse = pl.pallas_call(
        _lse_kernel,
        grid=(NB,),
        in_specs=[
            pl.BlockSpec((BBLK, L, PAIR), lambda i: (i, 0, 0)),
            pl.BlockSpec((BBLK, L), lambda i: (i, 0)),
            pl.BlockSpec((EMB, C), lambda i: (0, 0)),
        ],
        out_specs=pl.BlockSpec((L, C), lambda i: (0, 0)),
        out_shape=jax.ShapeDtypeStruct((L, C), jnp.float32),
        scratch_shapes=[
            pltpu.VMEM((L, C), jnp.float32),
            pltpu.VMEM((L, C), jnp.float32),
        ],
    )(emb3, parity, wt)

    out = pl.pallas_call(
        _out_kernel,
        grid=(NB,),
        in_specs=[
            pl.BlockSpec((BBLK, L, PAIR), lambda i: (i, 0, 0)),
            pl.BlockSpec((BBLK, L), lambda i: (i, 0)),
            pl.BlockSpec((EMB, C), lambda i: (0, 0)),
            pl.BlockSpec((L, C), lambda i: (0, 0)),
        ],
        out_specs=pl.BlockSpec((BBLK, L, C), lambda i: (i, 0, 0)),
        out_shape=jax.ShapeDtypeStruct((B, L, C), jnp.float32),
    )(emb3, parity, wt, lse)
    return out
